# baseline (speedup 1.0000x reference)
import jax
import jax.numpy as jnp
from jax.experimental import pallas as pl

SPIKE_THRESHOLD = 0.01
_BLK = 2048


def _fused_block(x_ref, w2_ref, b2_ref, o_ref):
    xb = x_ref[...]
    y = jnp.dot(xb, w2_ref[...], preferred_element_type=jnp.float32) + b2_ref[...]
    ax = jnp.abs(xb)
    m_even = jnp.max(ax[:, :64], axis=1, keepdims=True)
    m_odd = jnp.max(ax[:, 64:], axis=1, keepdims=True)
    n, w = xb.shape
    lane = jax.lax.broadcasted_iota(jnp.int32, (n, w), 1)
    peak = jnp.where(
        lane < 64,
        jnp.broadcast_to(m_even, (n, w)),
        jnp.broadcast_to(m_odd, (n, w)),
    )
    o_ref[...] = jnp.where(peak > SPIKE_THRESHOLD, y, 0.0)


def kernel(x, W, b):
    B, T, S, D = x.shape
    n2 = (B * T * S) // 2
    xf = x.reshape(n2, 2 * D)
    wt = W.T
    zero = jnp.zeros((D, D), dtype=W.dtype)
    w2 = jnp.block([[wt, zero], [zero, wt]])
    b2 = jnp.concatenate([b, b]).reshape(1, 2 * D)
    out = pl.pallas_call(
        _fused_block,
        grid=(n2 // _BLK,),
        in_specs=[
            pl.BlockSpec((_BLK, 2 * D), lambda i: (i, 0)),
            pl.BlockSpec((2 * D, 2 * D), lambda i: (0, 0)),
            pl.BlockSpec((1, 2 * D), lambda i: (0, 0)),
        ],
        out_specs=pl.BlockSpec((_BLK, 2 * D), lambda i: (i, 0)),
        out_shape=jax.ShapeDtypeStruct((n2, 2 * D), x.dtype),
    )(xf, w2, b2)
    return out.reshape(B, T, S, D)


# speedup vs baseline: 1.8231x; 1.8231x over previous
import jax
import jax.numpy as jnp
from jax.experimental import pallas as pl

SPIKE_THRESHOLD = 0.01
_BLK = 4096


def _fused_block(x_ref, wt_ref, b_ref, o_ref):
    xb = x_ref[...]
    y = jnp.dot(xb, wt_ref[...], preferred_element_type=jnp.float32) + b_ref[...]
    spike = (jnp.abs(xb) > SPIKE_THRESHOLD).any(axis=1, keepdims=True)
    o_ref[...] = jnp.where(spike, y, 0.0)


def kernel(x, W, b):
    B, T, S, D = x.shape
    n = B * T * S
    xf = x.reshape(n, D)
    out = pl.pallas_call(
        _fused_block,
        grid=(n // _BLK,),
        in_specs=[
            pl.BlockSpec((_BLK, D), lambda i: (i, 0)),
            pl.BlockSpec((D, D), lambda i: (0, 0)),
            pl.BlockSpec((1, D), lambda i: (0, 0)),
        ],
        out_specs=pl.BlockSpec((_BLK, D), lambda i: (i, 0)),
        out_shape=jax.ShapeDtypeStruct((n, D), x.dtype),
    )(xf, W.T, b.reshape(1, D))
    return out.reshape(B, T, S, D)


# speedup vs baseline: 5.3915x; 2.9574x over previous
import jax
import jax.numpy as jnp
from jax.experimental import pallas as pl

SPIKE_THRESHOLD = 0.01


def _fused_panel(x_ref, w_ref, b_ref, o_ref):
    xb = x_ref[0]
    y = jnp.dot(w_ref[...], xb, preferred_element_type=jnp.float32) + b_ref[...]
    peak = jnp.max(jnp.abs(xb), axis=0, keepdims=True)
    o_ref[0] = jnp.where(peak > SPIKE_THRESHOLD, y, 0.0)


def kernel(x, W, b):
    B, T, S, D = x.shape
    nbt = B * T
    xt = x.transpose(0, 1, 3, 2).reshape(nbt, D, S)
    out_t = pl.pallas_call(
        _fused_panel,
        grid=(nbt,),
        in_specs=[
            pl.BlockSpec((1, D, S), lambda i: (i, 0, 0)),
            pl.BlockSpec((D, D), lambda i: (0, 0)),
            pl.BlockSpec((D, 1), lambda i: (0, 0)),
        ],
        out_specs=pl.BlockSpec((1, D, S), lambda i: (i, 0, 0)),
        out_shape=jax.ShapeDtypeStruct((nbt, D, S), x.dtype),
    )(xt, W, b.reshape(D, 1))
    return out_t.reshape(B, T, D, S).transpose(0, 1, 3, 2)


# speedup vs baseline: 7.5000x; 1.3911x over previous
import jax
import jax.numpy as jnp
from jax.experimental import pallas as pl

SPIKE_THRESHOLD = 0.01


_BT_BLK = 2


def _fused_panel(x_ref, w_ref, b_ref, o_ref):
    for p in range(x_ref.shape[0]):
        xb = x_ref[p]
        y = jnp.dot(w_ref[...], xb, preferred_element_type=jnp.float32) + b_ref[...]
        peak = jnp.max(jnp.abs(xb), axis=0, keepdims=True)
        o_ref[p] = jnp.where(peak > SPIKE_THRESHOLD, y, 0.0)


def kernel(x, W, b):
    B, T, S, D = x.shape
    nbt = B * T
    xt = x.transpose(0, 1, 3, 2).reshape(nbt, D, S)
    out_t = pl.pallas_call(
        _fused_panel,
        grid=(nbt // _BT_BLK,),
        in_specs=[
            pl.BlockSpec((_BT_BLK, D, S), lambda i: (i, 0, 0)),
            pl.BlockSpec((D, D), lambda i: (0, 0)),
            pl.BlockSpec((D, 1), lambda i: (0, 0)),
        ],
        out_specs=pl.BlockSpec((_BT_BLK, D, S), lambda i: (i, 0, 0)),
        out_shape=jax.ShapeDtypeStruct((nbt, D, S), x.dtype),
    )(xt, W, b.reshape(D, 1))
    return out_t.reshape(B, T, D, S).transpose(0, 1, 3, 2)


# speedup vs baseline: 8.6877x; 1.1584x over previous
import jax
import jax.numpy as jnp
from jax.experimental import pallas as pl

SPIKE_THRESHOLD = 0.01


_BT_BLK = 4


def _fused_panel(x_ref, w_ref, b_ref, o_ref):
    for p in range(x_ref.shape[0]):
        xb = x_ref[p]
        y = jnp.dot(w_ref[...], xb, preferred_element_type=jnp.float32) + b_ref[...]
        peak = jnp.max(jnp.abs(xb), axis=0, keepdims=True)
        o_ref[p] = jnp.where(peak > SPIKE_THRESHOLD, y, 0.0)


def kernel(x, W, b):
    B, T, S, D = x.shape
    nbt = B * T
    xt = x.transpose(0, 1, 3, 2).reshape(nbt, D, S)
    out_t = pl.pallas_call(
        _fused_panel,
        grid=(nbt // _BT_BLK,),
        in_specs=[
            pl.BlockSpec((_BT_BLK, D, S), lambda i: (i, 0, 0)),
            pl.BlockSpec((D, D), lambda i: (0, 0)),
            pl.BlockSpec((D, 1), lambda i: (0, 0)),
        ],
        out_specs=pl.BlockSpec((_BT_BLK, D, S), lambda i: (i, 0, 0)),
        out_shape=jax.ShapeDtypeStruct((nbt, D, S), x.dtype),
    )(xt, W, b.reshape(D, 1))
    return out_t.reshape(B, T, D, S).transpose(0, 1, 3, 2)


# speedup vs baseline: 8.8962x; 1.0240x over previous
import jax
import jax.numpy as jnp
from jax.experimental import pallas as pl

SPIKE_THRESHOLD = 0.01


_BT_BLK = 8


def _fused_panel(x_ref, w_ref, b_ref, o_ref):
    for p in range(x_ref.shape[0]):
        xb = x_ref[p]
        y = jnp.dot(w_ref[...], xb, preferred_element_type=jnp.float32) + b_ref[...]
        peak = jnp.max(jnp.abs(xb), axis=0, keepdims=True)
        o_ref[p] = jnp.where(peak > SPIKE_THRESHOLD, y, 0.0)


def kernel(x, W, b):
    B, T, S, D = x.shape
    nbt = B * T
    xt = x.transpose(0, 1, 3, 2).reshape(nbt, D, S)
    out_t = pl.pallas_call(
        _fused_panel,
        grid=(nbt // _BT_BLK,),
        in_specs=[
            pl.BlockSpec((_BT_BLK, D, S), lambda i: (i, 0, 0)),
            pl.BlockSpec((D, D), lambda i: (0, 0)),
            pl.BlockSpec((D, 1), lambda i: (0, 0)),
        ],
        out_specs=pl.BlockSpec((_BT_BLK, D, S), lambda i: (i, 0, 0)),
        out_shape=jax.ShapeDtypeStruct((nbt, D, S), x.dtype),
    )(xt, W, b.reshape(D, 1))
    return out_t.reshape(B, T, D, S).transpose(0, 1, 3, 2)
